# baseline (speedup 1.0000x reference)
import functools

import jax
import jax.numpy as jnp
from jax import lax
from jax.experimental import pallas as pl
from jax.experimental.pallas import tpu as pltpu
from jax.experimental.pallas import tpu_sc as plsc

VOCAB = 1000000
EMBED_DIM = 16
BATCH = 16384
HIST_LEN = 200

NC = 2
NS = 16
NW = NC * NS
ROWS_PER_W = BATCH // NW
RCHUNK = 16
NCHUNKS = ROWS_PER_W // RCHUNK
LOOKUPS = RCHUNK * HIST_LEN
JBLK = 20
NJB = HIST_LEN // JBLK


def _sc_body(x_hbm, tbl_hbm, w_hbm, out_hbm, idx_v, vals_v, w_v, out_v, sem):
    c = lax.axis_index("c")
    s = lax.axis_index("s")
    wid = s * NC + c
    base = wid * ROWS_PER_W
    pltpu.sync_copy(w_hbm, w_v)
    wvec = w_v[...]
    scale = jnp.float32(1.0 / HIST_LEN)
    lane = lax.iota(jnp.int32, 16)

    def chunk(i, carry):
        row0 = base + i * RCHUNK
        pltpu.sync_copy(x_hbm.at[pl.ds(row0 * HIST_LEN, LOOKUPS)], idx_v)
        pltpu.async_copy(tbl_hbm.at[idx_v], vals_v, sem).wait()

        def jblk(jb, accs):
            off = jb * JBLK
            new = []
            for r in range(RCHUNK):
                a = accs[r]
                for j in range(JBLK):
                    a = a + vals_v[r * HIST_LEN + off + j, pl.ds(0, 16)]
                new.append(a)
            return tuple(new)

        zero = jnp.zeros((16,), jnp.float32)
        accs = lax.fori_loop(0, NJB, jblk, (zero,) * RCHUNK)
        tot = zero
        for r in range(RCHUNK):
            a = accs[r] * wvec
            for sh in (8, 4, 2, 1):
                a = a + a.at[(lane + sh) % 16].get(mode="promise_in_bounds")
            tot = jnp.where(lane == r, a, tot)
        out_v[pl.ds(i * RCHUNK, RCHUNK)] = tot * scale
        return carry

    lax.fori_loop(0, NCHUNKS, chunk, 0)
    pltpu.sync_copy(out_v, out_hbm.at[pl.ds(base, ROWS_PER_W)])


_sc_embed_pool = functools.partial(
    pl.kernel,
    out_type=jax.ShapeDtypeStruct((BATCH,), jnp.float32),
    mesh=plsc.VectorSubcoreMesh(core_axis_name="c", subcore_axis_name="s"),
    compiler_params=pltpu.CompilerParams(use_tc_tiling_on_sc=False),
    scratch_types=[
        pltpu.VMEM((LOOKUPS,), jnp.int32),
        pltpu.VMEM((LOOKUPS, EMBED_DIM), jnp.float32),
        pltpu.VMEM((EMBED_DIM,), jnp.float32),
        pltpu.VMEM((ROWS_PER_W,), jnp.float32),
        pltpu.SemaphoreType.DMA,
    ],
)(_sc_body)


@jax.jit
def kernel(x, table, W):
    xf = x.astype(jnp.int32).reshape(BATCH * HIST_LEN)
    out = _sc_embed_pool(xf, table, W.reshape(EMBED_DIM))
    return out.reshape(BATCH, 1)


# speedup vs baseline: 1.0824x; 1.0824x over previous
import functools

import jax
import jax.numpy as jnp
from jax import lax
from jax.experimental import pallas as pl
from jax.experimental.pallas import tpu as pltpu
from jax.experimental.pallas import tpu_sc as plsc

VOCAB = 1000000
EMBED_DIM = 16
BATCH = 16384
HIST_LEN = 200

NC = 2
NS = 16
NW = NC * NS
ROWS_PER_W = BATCH // NW
RCHUNK = 16
NCHUNKS = ROWS_PER_W // RCHUNK
NFULL = HIST_LEN // 16
NTAIL = HIST_LEN - NFULL * 16


def _vtab_body(t_ref, w_ref, o_ref):
    o_ref[...] = jnp.dot(t_ref[...], w_ref[...],
                         preferred_element_type=jnp.float32)


def _compute_vtab(table, W):
    tbl2 = table.reshape(VOCAB // 8, 128)
    wbd = (jnp.eye(8, dtype=jnp.float32)[:, None, :]
           * W.reshape(EMBED_DIM)[None, :, None]).reshape(128, 8)
    rows = VOCAB // 8
    block = 5000
    v2 = pl.pallas_call(
        _vtab_body,
        grid=(rows // block,),
        in_specs=[
            pl.BlockSpec((block, 128), lambda i: (i, 0)),
            pl.BlockSpec((128, 8), lambda i: (0, 0)),
        ],
        out_specs=pl.BlockSpec((block, 8), lambda i: (i, 0)),
        out_shape=jax.ShapeDtypeStruct((rows, 8), jnp.float32),
    )(tbl2, wbd)
    return v2.reshape(VOCAB)


def _sc_body(x_hbm, v_hbm, out_hbm, idx_v, vals_v, out_v, sem):
    c = lax.axis_index("c")
    s = lax.axis_index("s")
    wid = s * NC + c
    base = wid * ROWS_PER_W
    scale = jnp.float32(1.0 / HIST_LEN)
    lane = lax.iota(jnp.int32, 16)
    tailmask = lane < NTAIL

    def chunk(i, carry):
        row0 = base + i * RCHUNK
        for r in range(RCHUNK):
            pltpu.async_copy(x_hbm.at[row0 + r],
                             idx_v.at[pl.ds(r * 256, HIST_LEN)], sem)
        for r in range(RCHUNK):
            pltpu.make_async_copy(x_hbm.at[row0 + r],
                                  idx_v.at[pl.ds(r * 256, HIST_LEN)],
                                  sem).wait()
        for r in range(RCHUNK):
            pltpu.async_copy(v_hbm.at[idx_v.at[pl.ds(r * 256, HIST_LEN)]],
                             vals_v.at[pl.ds(r * HIST_LEN, HIST_LEN)], sem)
        for r in range(RCHUNK):
            pltpu.make_async_copy(v_hbm.at[idx_v.at[pl.ds(r * 256, HIST_LEN)]],
                                  vals_v.at[pl.ds(r * HIST_LEN, HIST_LEN)],
                                  sem).wait()
        tot = jnp.zeros((16,), jnp.float32)
        for r in range(RCHUNK):
            a = jnp.zeros((16,), jnp.float32)
            for k in range(NFULL):
                a = a + vals_v[pl.ds(r * HIST_LEN + k * 16, 16)]
            tail = vals_v[pl.ds(r * HIST_LEN + NFULL * 16, 16)]
            a = a + jnp.where(tailmask, tail, jnp.float32(0.0))
            for sh in (8, 4, 2, 1):
                a = a + a.at[(lane + sh) % 16].get(mode="promise_in_bounds")
            tot = jnp.where(lane == r, a, tot)
        out_v[pl.ds(i * RCHUNK, RCHUNK)] = tot * scale
        return carry

    lax.fori_loop(0, NCHUNKS, chunk, 0)
    pltpu.sync_copy(out_v, out_hbm.at[pl.ds(base, ROWS_PER_W)])


_sc_gather_pool = functools.partial(
    pl.kernel,
    out_type=jax.ShapeDtypeStruct((BATCH,), jnp.float32),
    mesh=plsc.VectorSubcoreMesh(core_axis_name="c", subcore_axis_name="s"),
    compiler_params=pltpu.CompilerParams(use_tc_tiling_on_sc=False),
    scratch_types=[
        pltpu.VMEM((RCHUNK * 256,), jnp.int32),
        pltpu.VMEM((RCHUNK * HIST_LEN + 16,), jnp.float32),
        pltpu.VMEM((ROWS_PER_W,), jnp.float32),
        pltpu.SemaphoreType.DMA,
    ],
)(_sc_body)


@jax.jit
def kernel(x, table, W):
    v = _compute_vtab(table, W)
    out = _sc_gather_pool(x.astype(jnp.int32), v)
    return out.reshape(BATCH, 1)


# speedup vs baseline: 2.5503x; 2.3562x over previous
import functools

import jax
import jax.numpy as jnp
from jax import lax
from jax.experimental import pallas as pl
from jax.experimental.pallas import tpu as pltpu
from jax.experimental.pallas import tpu_sc as plsc

VOCAB = 1000000
EMBED_DIM = 16
BATCH = 16384
HIST_LEN = 200

NC = 2
NS = 16
NW = NC * NS
GPW = (BATCH // 128) // NW
LT = HIST_LEN // 8


def _v_body(t_ref, w_ref, o_ref):
    acc = jnp.zeros((t_ref.shape[1],), jnp.float32)
    for d in range(EMBED_DIM):
        acc = acc + t_ref[d, :] * w_ref[0, d]
    o_ref[...] = acc


def _compute_v(tableT, W):
    blk = 16384
    grid = (VOCAB + blk - 1) // blk
    return pl.pallas_call(
        _v_body,
        grid=(grid,),
        in_specs=[
            pl.BlockSpec((EMBED_DIM, blk), lambda i: (0, i)),
            pl.BlockSpec(memory_space=pltpu.SMEM),
        ],
        out_specs=pl.BlockSpec((blk,), lambda i: (i,)),
        out_shape=jax.ShapeDtypeStruct((VOCAB,), jnp.float32),
    )(tableT, W)


def _sc_body(xt_hbm, v_hbm, out_hbm, idx_v, vals_v, out_v, sem):
    c = lax.axis_index("c")
    s = lax.axis_index("s")
    wid = s * NC + c
    scale = jnp.float32(1.0 / HIST_LEN)
    zero = jnp.zeros((16,), jnp.float32)

    for g in range(GPW):
        col = pl.multiple_of((wid * GPW + g) * 128, 128)

        def ltile(li, accs):
            r0 = pl.multiple_of(li * 8, 8)
            pltpu.sync_copy(xt_hbm.at[pl.ds(r0, 8), pl.ds(col, 128)], idx_v)
            for t in range(8):
                pltpu.async_copy(v_hbm.at[idx_v.at[t]], vals_v.at[t], sem)
            for t in range(8):
                pltpu.make_async_copy(v_hbm.at[idx_v.at[t]], vals_v.at[t],
                                      sem).wait()
            new = []
            for j in range(8):
                a = accs[j]
                for t in range(8):
                    a = a + vals_v[t, pl.ds(j * 16, 16)]
                new.append(a)
            return tuple(new)

        accs = lax.fori_loop(0, LT, ltile, (zero,) * 8)
        for j in range(8):
            out_v[pl.ds(j * 16, 16)] = accs[j] * scale
        pltpu.sync_copy(out_v, out_hbm.at[pl.ds(col, 128)])


_sc_gather_pool = functools.partial(
    pl.kernel,
    out_type=jax.ShapeDtypeStruct((BATCH,), jnp.float32),
    mesh=plsc.VectorSubcoreMesh(core_axis_name="c", subcore_axis_name="s"),
    scratch_types=[
        pltpu.VMEM((8, 128), jnp.int32),
        pltpu.VMEM((8, 128), jnp.float32),
        pltpu.VMEM((128,), jnp.float32),
        pltpu.SemaphoreType.DMA,
    ],
)(_sc_body)


@jax.jit
def kernel(x, table, W):
    v = _compute_v(table.T, W.astype(jnp.float32))
    out = _sc_gather_pool(x.astype(jnp.int32).T, v)
    return out.reshape(BATCH, 1)


# speedup vs baseline: 4.4169x; 1.7319x over previous
import functools

import jax
import jax.numpy as jnp
from jax import lax
from jax.experimental import pallas as pl
from jax.experimental.pallas import tpu as pltpu
from jax.experimental.pallas import tpu_sc as plsc

VOCAB = 1000000
EMBED_DIM = 16
BATCH = 16384
HIST_LEN = 200

NC = 2
NS = 16
NW = NC * NS
GPW = (BATCH // 128) // NW
LT = HIST_LEN // 8


def _v_body(t_ref, w_ref, o_ref):
    acc = jnp.zeros((t_ref.shape[1],), jnp.float32)
    for d in range(EMBED_DIM):
        acc = acc + t_ref[d, :] * w_ref[0, d]
    o_ref[...] = acc


def _compute_v(tableT, W):
    blk = 32768
    grid = (VOCAB + blk - 1) // blk
    return pl.pallas_call(
        _v_body,
        grid=(grid,),
        in_specs=[
            pl.BlockSpec((EMBED_DIM, blk), lambda i: (0, i)),
            pl.BlockSpec(memory_space=pltpu.SMEM),
        ],
        out_specs=pl.BlockSpec((blk,), lambda i: (i,)),
        out_shape=jax.ShapeDtypeStruct((VOCAB,), jnp.float32),
    )(tableT, W)


def _gathers(v_hbm, idx_v, li, vals_v, vb, sem, fire):
    for t in range(8):
        cp = pltpu.make_async_copy(v_hbm.at[idx_v.at[li, t]],
                                   vals_v.at[vb, t], sem)
        if fire:
            cp.start()
        else:
            cp.wait()


def _sc_body(xt_hbm, v_hbm, out_hbm, idx_v, vals_v, out_v, semi, sg0, sg1):
    c = lax.axis_index("c")
    s = lax.axis_index("s")
    wid = s * NC + c
    scale = jnp.float32(1.0 / HIST_LEN)
    zero = jnp.zeros((16,), jnp.float32)

    for g in range(GPW):
        col = pl.multiple_of((wid * GPW + g) * 128, 128)
        for li in range(LT):
            pltpu.async_copy(
                xt_hbm.at[pl.ds(li * 8, 8), pl.ds(col, 128)],
                idx_v.at[li], semi)
        for li in range(LT):
            pltpu.make_async_copy(
                xt_hbm.at[pl.ds(li * 8, 8), pl.ds(col, 128)],
                idx_v.at[li], semi).wait()
        _gathers(v_hbm, idx_v, 0, vals_v, 0, sg0, True)

        def pair(k, accs):
            new = list(accs)
            _gathers(v_hbm, idx_v, 2 * k + 1, vals_v, 1, sg1, True)
            _gathers(v_hbm, idx_v, 2 * k, vals_v, 0, sg0, False)
            for j in range(8):
                a = new[j]
                for t in range(8):
                    a = a + vals_v[0, t, pl.ds(j * 16, 16)]
                new[j] = a
            _gathers(v_hbm, idx_v, 2 * k + 2, vals_v, 0, sg0, True)
            _gathers(v_hbm, idx_v, 2 * k + 1, vals_v, 1, sg1, False)
            for j in range(8):
                a = new[j]
                for t in range(8):
                    a = a + vals_v[1, t, pl.ds(j * 16, 16)]
                new[j] = a
            return tuple(new)

        accs = lax.fori_loop(0, (LT - 1) // 2, pair, (zero,) * 8)
        _gathers(v_hbm, idx_v, LT - 1, vals_v, 0, sg0, False)
        for j in range(8):
            a = accs[j]
            for t in range(8):
                a = a + vals_v[0, t, pl.ds(j * 16, 16)]
            out_v[pl.ds(j * 16, 16)] = a * scale
        pltpu.sync_copy(out_v, out_hbm.at[pl.ds(col, 128)])


_sc_gather_pool = functools.partial(
    pl.kernel,
    out_type=jax.ShapeDtypeStruct((BATCH,), jnp.float32),
    mesh=plsc.VectorSubcoreMesh(core_axis_name="c", subcore_axis_name="s"),
    scratch_types=[
        pltpu.VMEM((LT, 8, 128), jnp.int32),
        pltpu.VMEM((2, 8, 128), jnp.float32),
        pltpu.VMEM((128,), jnp.float32),
        pltpu.SemaphoreType.DMA,
        pltpu.SemaphoreType.DMA,
        pltpu.SemaphoreType.DMA,
    ],
)(_sc_body)


@jax.jit
def kernel(x, table, W):
    v = _compute_v(table.T, W.astype(jnp.float32))
    out = _sc_gather_pool(x.astype(jnp.int32).T, v)
    return out.reshape(BATCH, 1)


# speedup vs baseline: 7.8982x; 1.7882x over previous
import functools

import jax
import jax.numpy as jnp
from jax import lax
from jax.experimental import pallas as pl
from jax.experimental.pallas import tpu as pltpu
from jax.experimental.pallas import tpu_sc as plsc

VOCAB = 1000000
EMBED_DIM = 16
BATCH = 16384
HIST_LEN = 200

NC = 2
NS = 16
NW = NC * NS
GPW = (BATCH // 128) // NW
LT = HIST_LEN // 8


def _v_body(t_ref, w_ref, o_ref):
    acc = jnp.zeros((t_ref.shape[1],), jnp.float32)
    for d in range(EMBED_DIM):
        acc = acc + t_ref[d, :] * w_ref[0, d]
    o_ref[...] = acc


def _compute_v(tableT, W):
    blk = 32768
    grid = (VOCAB + blk - 1) // blk
    return pl.pallas_call(
        _v_body,
        grid=(grid,),
        in_specs=[
            pl.BlockSpec((EMBED_DIM, blk), lambda i: (0, i)),
            pl.BlockSpec(memory_space=pltpu.SMEM),
        ],
        out_specs=pl.BlockSpec((blk,), lambda i: (i,)),
        out_shape=jax.ShapeDtypeStruct((VOCAB,), jnp.float32),
    )(tableT, W)


def _gathers(v_hbm, idx_v, li, vals_v, vb, sem, fire):
    for t in range(8):
        cp = pltpu.make_async_copy(v_hbm.at[idx_v.at[li, t]],
                                   vals_v.at[vb, t], sem)
        if fire:
            cp.start()
        else:
            cp.wait()


def _sc_body(xt_hbm, v_hbm, out_hbm, idx_v, vals_v, out_v, bounce, vsh, semi,
             sg0, sg1):
    c = lax.axis_index("c")
    s = lax.axis_index("s")
    wid = s * NC + c
    scale = jnp.float32(1.0 / HIST_LEN)
    zero = jnp.zeros((16,), jnp.float32)

    sbase = pl.multiple_of(s * 62528, 64)
    for k in range(4):
        koff = pl.multiple_of(sbase + k * 16384, 64)

        @pl.when((s < NS - 1) | (k < 3))
        def _stage_main():
            n = 16384 if k < 3 else 13376
            pltpu.sync_copy(v_hbm.at[pl.ds(koff, n)], bounce.at[pl.ds(0, n)])
            pltpu.sync_copy(bounce.at[pl.ds(0, n)], vsh.at[pl.ds(koff, n)])

        @pl.when((s == NS - 1) & (k == 3))
        def _stage_tail():
            pltpu.sync_copy(v_hbm.at[pl.ds(koff, 13056)],
                            bounce.at[pl.ds(0, 13056)])
            pltpu.sync_copy(bounce.at[pl.ds(0, 13056)],
                            vsh.at[pl.ds(koff, 13056)])

    plsc.subcore_barrier()

    for g in range(GPW):
        col = pl.multiple_of((wid * GPW + g) * 128, 128)
        for li in range(LT):
            pltpu.async_copy(
                xt_hbm.at[pl.ds(li * 8, 8), pl.ds(col, 128)],
                idx_v.at[li], semi)
        for li in range(LT):
            pltpu.make_async_copy(
                xt_hbm.at[pl.ds(li * 8, 8), pl.ds(col, 128)],
                idx_v.at[li], semi).wait()
        _gathers(vsh, idx_v, 0, vals_v, 0, sg0, True)

        def pair(k, accs):
            new = list(accs)
            _gathers(vsh, idx_v, 2 * k + 1, vals_v, 1, sg1, True)
            _gathers(vsh, idx_v, 2 * k, vals_v, 0, sg0, False)
            for j in range(8):
                a = new[j]
                for t in range(8):
                    a = a + vals_v[0, t, pl.ds(j * 16, 16)]
                new[j] = a
            _gathers(vsh, idx_v, 2 * k + 2, vals_v, 0, sg0, True)
            _gathers(vsh, idx_v, 2 * k + 1, vals_v, 1, sg1, False)
            for j in range(8):
                a = new[j]
                for t in range(8):
                    a = a + vals_v[1, t, pl.ds(j * 16, 16)]
                new[j] = a
            return tuple(new)

        accs = lax.fori_loop(0, (LT - 1) // 2, pair, (zero,) * 8)
        _gathers(vsh, idx_v, LT - 1, vals_v, 0, sg0, False)
        for j in range(8):
            a = accs[j]
            for t in range(8):
                a = a + vals_v[0, t, pl.ds(j * 16, 16)]
            out_v[pl.ds(j * 16, 16)] = a * scale
        pltpu.sync_copy(out_v, out_hbm.at[pl.ds(col, 128)])


_sc_gather_pool = functools.partial(
    pl.kernel,
    out_type=jax.ShapeDtypeStruct((BATCH,), jnp.float32),
    mesh=plsc.VectorSubcoreMesh(core_axis_name="c", subcore_axis_name="s"),
    scratch_types=[
        pltpu.VMEM((LT, 8, 128), jnp.int32),
        pltpu.VMEM((2, 8, 128), jnp.float32),
        pltpu.VMEM((128,), jnp.float32),
        pltpu.VMEM((16384,), jnp.float32),
        pltpu.VMEM_SHARED((VOCAB,), jnp.float32),
        pltpu.SemaphoreType.DMA,
        pltpu.SemaphoreType.DMA,
        pltpu.SemaphoreType.DMA,
    ],
)(_sc_body)


@jax.jit
def kernel(x, table, W):
    v = _compute_v(table.T, W.astype(jnp.float32))
    out = _sc_gather_pool(x.astype(jnp.int32).T, v)
    return out.reshape(BATCH, 1)


# speedup vs baseline: 8.5741x; 1.0856x over previous
import functools

import jax
import jax.numpy as jnp
from jax import lax
from jax.experimental import pallas as pl
from jax.experimental.pallas import tpu as pltpu
from jax.experimental.pallas import tpu_sc as plsc

VOCAB = 1000000
EMBED_DIM = 16
BATCH = 16384
HIST_LEN = 200

NC = 2
NS = 16
NW = NC * NS
GPW = (BATCH // 128) // NW
LT = HIST_LEN // 8


def _v_body(t_ref, w_ref, o_ref):
    acc = jnp.zeros((t_ref.shape[1],), jnp.float32)
    for d in range(EMBED_DIM):
        acc = acc + t_ref[d, :] * w_ref[0, d]
    o_ref[...] = acc


def _compute_v(tableT, W):
    blk = 65536
    grid = (VOCAB + blk - 1) // blk
    return pl.pallas_call(
        _v_body,
        grid=(grid,),
        in_specs=[
            pl.BlockSpec((EMBED_DIM, blk), lambda i: (0, i)),
            pl.BlockSpec(memory_space=pltpu.SMEM),
        ],
        out_specs=pl.BlockSpec((blk,), lambda i: (i,)),
        out_shape=jax.ShapeDtypeStruct((VOCAB,), jnp.float32),
    )(tableT, W)


def _gathers(v_hbm, idx_v, li, vals_v, vb, sem, fire):
    for t in range(8):
        cp = pltpu.make_async_copy(v_hbm.at[idx_v.at[li, t]],
                                   vals_v.at[vb, t], sem)
        if fire:
            cp.start()
        else:
            cp.wait()


def _sc_body(xt_hbm, v_hbm, out_hbm, idx_v, vals_v, out_v, bounce, vsh, semi,
             sg0, sg1):
    c = lax.axis_index("c")
    s = lax.axis_index("s")
    wid = s * NC + c
    scale = jnp.float32(1.0 / HIST_LEN)
    zero = jnp.zeros((16,), jnp.float32)

    sbase = pl.multiple_of(s * 62528, 64)
    for k in range(4):
        koff = pl.multiple_of(sbase + k * 16384, 64)

        @pl.when((s < NS - 1) | (k < 3))
        def _stage_main():
            n = 16384 if k < 3 else 13376
            pltpu.sync_copy(v_hbm.at[pl.ds(koff, n)], bounce.at[pl.ds(0, n)])
            pltpu.sync_copy(bounce.at[pl.ds(0, n)], vsh.at[pl.ds(koff, n)])

        @pl.when((s == NS - 1) & (k == 3))
        def _stage_tail():
            pltpu.sync_copy(v_hbm.at[pl.ds(koff, 13056)],
                            bounce.at[pl.ds(0, 13056)])
            pltpu.sync_copy(bounce.at[pl.ds(0, 13056)],
                            vsh.at[pl.ds(koff, 13056)])

    plsc.subcore_barrier()

    for g in range(GPW):
        col = pl.multiple_of((wid * GPW + g) * 128, 128)
        for li in range(LT):
            pltpu.async_copy(
                xt_hbm.at[pl.ds(li * 8, 8), pl.ds(col, 128)],
                idx_v.at[li], semi)
        for li in range(LT):
            pltpu.make_async_copy(
                xt_hbm.at[pl.ds(li * 8, 8), pl.ds(col, 128)],
                idx_v.at[li], semi).wait()
        _gathers(vsh, idx_v, 0, vals_v, 0, sg0, True)

        def pair(k, accs):
            new = list(accs)
            _gathers(vsh, idx_v, 2 * k + 1, vals_v, 1, sg1, True)
            _gathers(vsh, idx_v, 2 * k, vals_v, 0, sg0, False)
            for j in range(8):
                a = new[j]
                for t in range(8):
                    a = a + vals_v[0, t, pl.ds(j * 16, 16)]
                new[j] = a
            _gathers(vsh, idx_v, 2 * k + 2, vals_v, 0, sg0, True)
            _gathers(vsh, idx_v, 2 * k + 1, vals_v, 1, sg1, False)
            for j in range(8):
                a = new[j]
                for t in range(8):
                    a = a + vals_v[1, t, pl.ds(j * 16, 16)]
                new[j] = a
            return tuple(new)

        accs = lax.fori_loop(0, (LT - 1) // 2, pair, (zero,) * 8)
        _gathers(vsh, idx_v, LT - 1, vals_v, 0, sg0, False)
        for j in range(8):
            a = accs[j]
            for t in range(8):
                a = a + vals_v[0, t, pl.ds(j * 16, 16)]
            out_v[pl.ds(j * 16, 16)] = a * scale
        pltpu.sync_copy(out_v, out_hbm.at[pl.ds(col, 128)])


_sc_gather_pool = functools.partial(
    pl.kernel,
    out_type=jax.ShapeDtypeStruct((BATCH,), jnp.float32),
    mesh=plsc.VectorSubcoreMesh(core_axis_name="c", subcore_axis_name="s"),
    scratch_types=[
        pltpu.VMEM((LT, 8, 128), jnp.int32),
        pltpu.VMEM((2, 8, 128), jnp.float32),
        pltpu.VMEM((128,), jnp.float32),
        pltpu.VMEM((16384,), jnp.float32),
        pltpu.VMEM_SHARED((VOCAB,), jnp.float32),
        pltpu.SemaphoreType.DMA,
        pltpu.SemaphoreType.DMA,
        pltpu.SemaphoreType.DMA,
    ],
)(_sc_body)


@jax.jit
def kernel(x, table, W):
    v = _compute_v(table.T, W.astype(jnp.float32))
    out = _sc_gather_pool(x.astype(jnp.int32).T, v)
    return out.reshape(BATCH, 1)


# speedup vs baseline: 8.8373x; 1.0307x over previous
import functools

import jax
import jax.numpy as jnp
from jax import lax
from jax.experimental import pallas as pl
from jax.experimental.pallas import tpu as pltpu
from jax.experimental.pallas import tpu_sc as plsc

VOCAB = 1000000
EMBED_DIM = 16
BATCH = 16384
HIST_LEN = 200

NC = 2
NS = 16
NW = NC * NS
GPW = (BATCH // 128) // NW
LT = HIST_LEN // 8


def _v_body(t_ref, w_ref, o_ref):
    acc = jnp.zeros((t_ref.shape[1],), jnp.float32)
    for d in range(EMBED_DIM):
        acc = acc + t_ref[d, :] * w_ref[0, d]
    o_ref[...] = acc


def _compute_v(tableT, W):
    blk = 131072
    grid = (VOCAB + blk - 1) // blk
    return pl.pallas_call(
        _v_body,
        grid=(grid,),
        in_specs=[
            pl.BlockSpec((EMBED_DIM, blk), lambda i: (0, i)),
            pl.BlockSpec(memory_space=pltpu.SMEM),
        ],
        out_specs=pl.BlockSpec((blk,), lambda i: (i,)),
        out_shape=jax.ShapeDtypeStruct((VOCAB,), jnp.float32),
    )(tableT, W)


def _gathers(v_hbm, idx_v, li, vals_v, vb, sem, fire):
    for t in range(8):
        cp = pltpu.make_async_copy(v_hbm.at[idx_v.at[li, t]],
                                   vals_v.at[vb, t], sem)
        if fire:
            cp.start()
        else:
            cp.wait()


def _sc_body(xt_hbm, v_hbm, out_hbm, idx_v, vals_v, out_v, bounce, vsh, semi,
             sg0, sg1):
    c = lax.axis_index("c")
    s = lax.axis_index("s")
    wid = s * NC + c
    scale = jnp.float32(1.0 / HIST_LEN)
    zero = jnp.zeros((16,), jnp.float32)

    sbase = pl.multiple_of(s * 62528, 64)
    for k in range(4):
        koff = pl.multiple_of(sbase + k * 16384, 64)

        @pl.when((s < NS - 1) | (k < 3))
        def _stage_main():
            n = 16384 if k < 3 else 13376
            pltpu.sync_copy(v_hbm.at[pl.ds(koff, n)], bounce.at[pl.ds(0, n)])
            pltpu.sync_copy(bounce.at[pl.ds(0, n)], vsh.at[pl.ds(koff, n)])

        @pl.when((s == NS - 1) & (k == 3))
        def _stage_tail():
            pltpu.sync_copy(v_hbm.at[pl.ds(koff, 13056)],
                            bounce.at[pl.ds(0, 13056)])
            pltpu.sync_copy(bounce.at[pl.ds(0, 13056)],
                            vsh.at[pl.ds(koff, 13056)])

    plsc.subcore_barrier()

    for g in range(GPW):
        col = pl.multiple_of((wid * GPW + g) * 128, 128)
        for li in range(LT):
            pltpu.async_copy(
                xt_hbm.at[pl.ds(li * 8, 8), pl.ds(col, 128)],
                idx_v.at[li], semi)
        for li in range(LT):
            pltpu.make_async_copy(
                xt_hbm.at[pl.ds(li * 8, 8), pl.ds(col, 128)],
                idx_v.at[li], semi).wait()
        _gathers(vsh, idx_v, 0, vals_v, 0, sg0, True)

        def pair(k, accs):
            new = list(accs)
            _gathers(vsh, idx_v, 2 * k + 1, vals_v, 1, sg1, True)
            _gathers(vsh, idx_v, 2 * k, vals_v, 0, sg0, False)
            for j in range(8):
                a = new[j]
                for t in range(8):
                    a = a + vals_v[0, t, pl.ds(j * 16, 16)]
                new[j] = a
            _gathers(vsh, idx_v, 2 * k + 2, vals_v, 0, sg0, True)
            _gathers(vsh, idx_v, 2 * k + 1, vals_v, 1, sg1, False)
            for j in range(8):
                a = new[j]
                for t in range(8):
                    a = a + vals_v[1, t, pl.ds(j * 16, 16)]
                new[j] = a
            return tuple(new)

        accs = lax.fori_loop(0, (LT - 1) // 2, pair, (zero,) * 8)
        _gathers(vsh, idx_v, LT - 1, vals_v, 0, sg0, False)
        for j in range(8):
            a = accs[j]
            for t in range(8):
                a = a + vals_v[0, t, pl.ds(j * 16, 16)]
            out_v[pl.ds(j * 16, 16)] = a * scale
        pltpu.sync_copy(out_v, out_hbm.at[pl.ds(col, 128)])


_sc_gather_pool = functools.partial(
    pl.kernel,
    out_type=jax.ShapeDtypeStruct((BATCH,), jnp.float32),
    mesh=plsc.VectorSubcoreMesh(core_axis_name="c", subcore_axis_name="s"),
    scratch_types=[
        pltpu.VMEM((LT, 8, 128), jnp.int32),
        pltpu.VMEM((2, 8, 128), jnp.float32),
        pltpu.VMEM((128,), jnp.float32),
        pltpu.VMEM((16384,), jnp.float32),
        pltpu.VMEM_SHARED((VOCAB,), jnp.float32),
        pltpu.SemaphoreType.DMA,
        pltpu.SemaphoreType.DMA,
        pltpu.SemaphoreType.DMA,
    ],
)(_sc_body)


@jax.jit
def kernel(x, table, W):
    v = _compute_v(table.T, W.astype(jnp.float32))
    out = _sc_gather_pool(x.astype(jnp.int32).T, v)
    return out.reshape(BATCH, 1)


# speedup vs baseline: 8.8653x; 1.0032x over previous
import functools

import jax
import jax.numpy as jnp
from jax import lax
from jax.experimental import pallas as pl
from jax.experimental.pallas import tpu as pltpu
from jax.experimental.pallas import tpu_sc as plsc

VOCAB = 1000000
EMBED_DIM = 16
BATCH = 16384
HIST_LEN = 200

NC = 2
NS = 16
NW = NC * NS
GPW = (BATCH // 128) // NW
LT = HIST_LEN // 8


def _v_body(t_ref, w_ref, o_ref):
    acc = jnp.zeros((t_ref.shape[1],), jnp.float32)
    for d in range(EMBED_DIM):
        acc = acc + t_ref[d, :] * w_ref[0, d]
    o_ref[...] = acc


def _compute_v(tableT, W):
    blk = 131072
    grid = (VOCAB + blk - 1) // blk
    return pl.pallas_call(
        _v_body,
        grid=(grid,),
        in_specs=[
            pl.BlockSpec((EMBED_DIM, blk), lambda i: (0, i)),
            pl.BlockSpec(memory_space=pltpu.SMEM),
        ],
        out_specs=pl.BlockSpec((blk,), lambda i: (i,)),
        out_shape=jax.ShapeDtypeStruct((VOCAB,), jnp.float32),
    )(tableT, W)


def _gathers(srcs, idx_v, li, vals_v, vb, sem, fire):
    src = srcs[1]
    for t in range(8):
        cp = pltpu.make_async_copy(src.at[idx_v.at[li, t]],
                                   vals_v.at[vb, t], sem)
        if fire:
            cp.start()
        else:
            cp.wait()


def _sc_body(xt_hbm, v_hbm, out_hbm, idx_v, vals_v, out_v, bounce, vsh, semi,
             sg0, sg1):
    c = lax.axis_index("c")
    s = lax.axis_index("s")
    wid = s * NC + c
    scale = jnp.float32(1.0 / HIST_LEN)
    zero = jnp.zeros((16,), jnp.float32)

    sbase = pl.multiple_of(s * 62528, 64)
    for k in range(4):
        koff = pl.multiple_of(sbase + k * 16384, 64)

        @pl.when((s < NS - 1) | (k < 3))
        def _stage_main():
            n = 16384 if k < 3 else 13376
            pltpu.sync_copy(v_hbm.at[pl.ds(koff, n)], bounce.at[pl.ds(0, n)])
            pltpu.sync_copy(bounce.at[pl.ds(0, n)], vsh.at[pl.ds(koff, n)])

        @pl.when((s == NS - 1) & (k == 3))
        def _stage_tail():
            pltpu.sync_copy(v_hbm.at[pl.ds(koff, 13056)],
                            bounce.at[pl.ds(0, 13056)])
            pltpu.sync_copy(bounce.at[pl.ds(0, 13056)],
                            vsh.at[pl.ds(koff, 13056)])

    plsc.subcore_barrier()

    for g in range(GPW):
        col = pl.multiple_of((wid * GPW + g) * 128, 128)
        for li in range(LT):
            pltpu.async_copy(
                xt_hbm.at[pl.ds(li * 8, 8), pl.ds(col, 128)],
                idx_v.at[li], semi)
        for li in range(LT):
            pltpu.make_async_copy(
                xt_hbm.at[pl.ds(li * 8, 8), pl.ds(col, 128)],
                idx_v.at[li], semi).wait()
        _gathers((v_hbm, vsh), idx_v, 0, vals_v, 0, sg0, True)

        def pair(k, accs):
            new = list(accs)
            _gathers((v_hbm, vsh), idx_v, 2 * k + 1, vals_v, 1, sg1, True)
            _gathers((v_hbm, vsh), idx_v, 2 * k, vals_v, 0, sg0, False)
            for j in range(8):
                a = new[j]
                for t in range(8):
                    a = a + vals_v[0, t, pl.ds(j * 16, 16)]
                new[j] = a
            _gathers((v_hbm, vsh), idx_v, 2 * k + 2, vals_v, 0, sg0, True)
            _gathers((v_hbm, vsh), idx_v, 2 * k + 1, vals_v, 1, sg1, False)
            for j in range(8):
                a = new[j]
                for t in range(8):
                    a = a + vals_v[1, t, pl.ds(j * 16, 16)]
                new[j] = a
            return tuple(new)

        accs = lax.fori_loop(0, (LT - 1) // 2, pair, (zero,) * 8)
        _gathers((v_hbm, vsh), idx_v, LT - 1, vals_v, 0, sg0, False)
        for j in range(8):
            a = accs[j]
            for t in range(8):
                a = a + vals_v[0, t, pl.ds(j * 16, 16)]
            out_v[pl.ds(j * 16, 16)] = a * scale
        pltpu.sync_copy(out_v, out_hbm.at[pl.ds(col, 128)])


_sc_gather_pool = functools.partial(
    pl.kernel,
    out_type=jax.ShapeDtypeStruct((BATCH,), jnp.float32),
    mesh=plsc.VectorSubcoreMesh(core_axis_name="c", subcore_axis_name="s"),
    scratch_types=[
        pltpu.VMEM((LT, 8, 128), jnp.int32),
        pltpu.VMEM((2, 8, 128), jnp.float32),
        pltpu.VMEM((128,), jnp.float32),
        pltpu.VMEM((16384,), jnp.float32),
        pltpu.VMEM_SHARED((VOCAB,), jnp.float32),
        pltpu.SemaphoreType.DMA,
        pltpu.SemaphoreType.DMA,
        pltpu.SemaphoreType.DMA,
    ],
)(_sc_body)


@jax.jit
def kernel(x, table, W):
    v = _compute_v(table.T, W.astype(jnp.float32))
    out = _sc_gather_pool(x.astype(jnp.int32).T, v)
    return out.reshape(BATCH, 1)
